# baseline (speedup 1.0000x reference)
import functools

import jax
import jax.numpy as jnp
from jax import lax
from jax.experimental import pallas as pl
from jax.experimental.pallas import tpu as pltpu
from jax.experimental.pallas import tpu_sc as plsc

_B, _P = 131072, 256
_L = 8
_NC, _NS, _LANES = 2, 16, 16
_NW = _NC * _NS
_SC_ROWS = 65536
_ROWS_PER_W = _SC_ROWS // _NW
_CH = 128
_NB = _CH // 8
_NCHUNK = _ROWS_PER_W // _CH

_mesh = plsc.VectorSubcoreMesh(core_axis_name="c", subcore_axis_name="s")


def _compute_chunk(in_v, w_vec, out_v):
    iota = lax.iota(jnp.int32, _LANES)
    for g in range(_CH // _LANES):
        rows = iota + (g * _LANES)
        band = rows // 8
        sub = rows % 8
        cols = [jnp.full((_LANES,), j, jnp.int32) for j in range(_L)]
        vs = [plsc.load_gather(in_v, [band, sub, cols[j]]) for j in range(_L)]
        sgn = [(-1.0 if j % 2 == 0 else 1.0) for j in range(_L)]
        cm = [vs[j] * sgn[j] for j in range(_L)]
        m = cm[0]
        for j in range(1, _L):
            m = jnp.maximum(m, cm[j])
        es = [jnp.exp(cm[j] - m) for j in range(_L)]
        s = es[0]
        for j in range(1, _L):
            s = s + es[j]
        scale = w_vec / s
        for j in range(_L):
            plsc.store_scatter(out_v, [rows, cols[j]], es[j] * (scale * sgn[j]))


@functools.partial(
    pl.kernel,
    mesh=_mesh,
    compiler_params=pltpu.CompilerParams(needs_layout_passes=False),
    out_type=jax.ShapeDtypeStruct((_SC_ROWS, _P), jnp.float32),
    scratch_types=[
        pltpu.VMEM((_NB, 8, 128), jnp.float32),
        pltpu.VMEM((_NB, 8, 128), jnp.float32),
        pltpu.VMEM((_LANES,), jnp.float32),
        pltpu.VMEM((_CH, _P), jnp.float32),
        pltpu.VMEM((_CH, _P), jnp.float32),
        pltpu.SemaphoreType.DMA,
        pltpu.SemaphoreType.DMA,
        pltpu.SemaphoreType.DMA,
        pltpu.SemaphoreType.DMA,
    ],
)
def _sc_kernel(in3_hbm, w_hbm, out_hbm,
               in_v0, in_v1, w_v, ov0, ov1, osem0, osem1, isem0, isem1):
    wid = lax.axis_index("s") * _NC + lax.axis_index("c")
    row0 = wid * _ROWS_PER_W

    pltpu.sync_copy(w_hbm, w_v)
    w_vec = w_v[...]

    zero = jnp.zeros((_LANES,), jnp.float32)

    def _zero_row(r, _):
        for c in range(_P // _LANES):
            ov0[r, pl.ds(c * _LANES, _LANES)] = zero
            ov1[r, pl.ds(c * _LANES, _LANES)] = zero
        return _

    lax.fori_loop(0, _CH, _zero_row, None)

    in_bufs = (in_v0, in_v1)
    out_bufs = (ov0, ov1)
    osems = (osem0, osem1)
    isems = (isem0, isem1)

    def _fetch(chunk, b):
        band0 = (row0 + chunk * _CH) // 8
        pltpu.async_copy(
            in3_hbm.at[pl.ds(band0, _NB), :, pl.ds(0, 128)],
            in_bufs[b], isems[b])

    def _fetch_wait(chunk, b):
        band0 = (row0 + chunk * _CH) // 8
        pltpu.make_async_copy(
            in3_hbm.at[pl.ds(band0, _NB), :, pl.ds(0, 128)],
            in_bufs[b], isems[b]).wait()

    _fetch(0, 0)

    def _pair(i, _):
        for b in range(2):
            chunk = i * 2 + b
            base = row0 + chunk * _CH

            @pl.when(chunk + 1 < _NCHUNK)
            def _():
                _fetch(chunk + 1, 1 - b)

            _fetch_wait(chunk, b)

            @pl.when(i > 0)
            def _():
                pltpu.make_async_copy(
                    out_bufs[b], out_hbm.at[pl.ds(0, _CH)], osems[b]).wait()

            _compute_chunk(in_bufs[b], w_vec, out_bufs[b])
            pltpu.async_copy(
                out_bufs[b], out_hbm.at[pl.ds(base, _CH)], osems[b])
        return _

    lax.fori_loop(0, _NCHUNK // 2, _pair, None)
    for b in range(2):
        pltpu.make_async_copy(
            out_bufs[b], out_hbm.at[pl.ds(0, _CH)], osems[b]).wait()


_TC_BLK = 2048


def _tc_body(w_ref, x_ref, o_ref):
    lane = lax.broadcasted_iota(jnp.int32, (1, _L), 1)
    signs = jnp.where(lane % 2 == 0, -1.0, 1.0)
    x = x_ref[:, 0:_L]
    cm = x * signs
    m = jnp.max(cm, axis=-1, keepdims=True)
    e = jnp.exp(cm - m)
    sm = e / jnp.sum(e, axis=-1, keepdims=True)
    delta = sm * signs * w_ref[0]
    blk = x_ref.shape[0]
    o_ref[...] = jnp.concatenate(
        [delta, jnp.zeros((blk, o_ref.shape[1] - _L), jnp.float32)], axis=1)


def _tc_kernel(inputs, clause_weight):
    nbot = _B - _SC_ROWS
    off = _SC_ROWS // _TC_BLK
    return pl.pallas_call(
        _tc_body,
        grid=(nbot // _TC_BLK,),
        in_specs=[
            pl.BlockSpec(memory_space=pltpu.SMEM),
            pl.BlockSpec((_TC_BLK, 128), lambda i: (i + off, 0)),
        ],
        out_specs=pl.BlockSpec((_TC_BLK, _P), lambda i: (i, 0)),
        out_shape=jax.ShapeDtypeStruct((nbot, _P), jnp.float32),
    )(clause_weight.reshape(1), inputs)


@jax.jit
def kernel(inputs, clause_weight):
    in3 = inputs.reshape(_B // 8, 8, _P)
    w16 = jnp.broadcast_to(clause_weight.reshape(()), (_LANES,))
    out_top = _sc_kernel(in3, w16)
    out_bot = _tc_kernel(inputs, clause_weight)
    return jnp.concatenate([out_top, out_bot], axis=0)


# speedup vs baseline: 1.7372x; 1.7372x over previous
import functools

import jax
import jax.numpy as jnp
from jax import lax
from jax.experimental import pallas as pl
from jax.experimental.pallas import tpu as pltpu
from jax.experimental.pallas import tpu_sc as plsc

_B, _P = 131072, 256
_L = 8
_NC, _NS, _LANES = 2, 16, 16
_NW = _NC * _NS
_ROWS_PER_W = _B // _NW
_CH = 128
_NB = _CH // 8
_NCHUNK = _ROWS_PER_W // _CH

_mesh = plsc.VectorSubcoreMesh(core_axis_name="c", subcore_axis_name="s")


def _compute_chunk(in_v, w_vec, out_v):
    iota = lax.iota(jnp.int32, _LANES)
    for g in range(_CH // _LANES):
        rows = iota + (g * _LANES)
        band = rows // 8
        sub = rows % 8
        cols = [jnp.full((_LANES,), j, jnp.int32) for j in range(_L)]
        vs = [plsc.load_gather(in_v, [band, sub, cols[j]]) for j in range(_L)]
        sgn = [(-1.0 if j % 2 == 0 else 1.0) for j in range(_L)]
        cm = [vs[j] * sgn[j] for j in range(_L)]
        m = cm[0]
        for j in range(1, _L):
            m = jnp.maximum(m, cm[j])
        es = [jnp.exp(cm[j] - m) for j in range(_L)]
        s = es[0]
        for j in range(1, _L):
            s = s + es[j]
        scale = w_vec / s
        for j in range(_L):
            plsc.store_scatter(out_v, [rows, cols[j]], es[j] * (scale * sgn[j]))


@functools.partial(
    pl.kernel,
    mesh=_mesh,
    compiler_params=pltpu.CompilerParams(
        needs_layout_passes=False, skip_device_barrier=True),
    out_type=jax.ShapeDtypeStruct((_B, _P), jnp.float32),
    scratch_types=[
        pltpu.VMEM((_NB, 8, 128), jnp.float32),
        pltpu.VMEM((_NB, 8, 128), jnp.float32),
        pltpu.VMEM((_LANES,), jnp.float32),
        pltpu.VMEM((_CH, _P), jnp.float32),
        pltpu.VMEM((_CH, _P), jnp.float32),
        pltpu.SemaphoreType.DMA,
        pltpu.SemaphoreType.DMA,
        pltpu.SemaphoreType.DMA,
        pltpu.SemaphoreType.DMA,
    ],
)
def _sc_kernel(in3_hbm, w_hbm, out_hbm,
               in_v0, in_v1, w_v, ov0, ov1, osem0, osem1, isem0, isem1):
    wid = lax.axis_index("s") * _NC + lax.axis_index("c")
    row0 = wid * _ROWS_PER_W

    pltpu.sync_copy(w_hbm, w_v)
    w_vec = w_v[...]

    zero = jnp.zeros((_LANES,), jnp.float32)

    def _zero_row(r, _):
        for c in range(_P // _LANES):
            ov0[r, pl.ds(c * _LANES, _LANES)] = zero
            ov1[r, pl.ds(c * _LANES, _LANES)] = zero
        return _

    lax.fori_loop(0, _CH, _zero_row, None)

    in_bufs = (in_v0, in_v1)
    out_bufs = (ov0, ov1)
    osems = (osem0, osem1)
    isems = (isem0, isem1)

    def _fetch(chunk, b):
        band0 = (row0 + chunk * _CH) // 8
        pltpu.async_copy(
            in3_hbm.at[pl.ds(band0, _NB), :, pl.ds(0, 128)],
            in_bufs[b], isems[b])

    def _fetch_wait(chunk, b):
        band0 = (row0 + chunk * _CH) // 8
        pltpu.make_async_copy(
            in3_hbm.at[pl.ds(band0, _NB), :, pl.ds(0, 128)],
            in_bufs[b], isems[b]).wait()

    _fetch(0, 0)

    def _pair(i, _):
        for b in range(2):
            chunk = i * 2 + b
            base = row0 + chunk * _CH

            @pl.when(chunk + 1 < _NCHUNK)
            def _():
                _fetch(chunk + 1, 1 - b)

            _fetch_wait(chunk, b)

            @pl.when(i > 0)
            def _():
                pltpu.make_async_copy(
                    out_bufs[b], out_hbm.at[pl.ds(0, _CH)], osems[b]).wait()

            _compute_chunk(in_bufs[b], w_vec, out_bufs[b])
            pltpu.async_copy(
                out_bufs[b], out_hbm.at[pl.ds(base, _CH)], osems[b])
        return _

    lax.fori_loop(0, _NCHUNK // 2, _pair, None)
    for b in range(2):
        pltpu.make_async_copy(
            out_bufs[b], out_hbm.at[pl.ds(0, _CH)], osems[b]).wait()


@jax.jit
def kernel(inputs, clause_weight):
    in3 = inputs.reshape(_B // 8, 8, _P)
    w16 = jnp.broadcast_to(clause_weight.reshape(()), (_LANES,))
    return _sc_kernel(in3, w16)


# speedup vs baseline: 1.7817x; 1.0257x over previous
import functools

import jax
import jax.numpy as jnp
from jax import lax
from jax.experimental import pallas as pl
from jax.experimental.pallas import tpu as pltpu
from jax.experimental.pallas import tpu_sc as plsc

_B, _P = 131072, 256
_L = 8
_NC, _NS, _LANES = 2, 16, 16
_NW = _NC * _NS
_ROWS_PER_W = _B // _NW
_CH = 128
_NB = _CH // 8
_NCHUNK = _ROWS_PER_W // _CH

_mesh = plsc.VectorSubcoreMesh(core_axis_name="c", subcore_axis_name="s")


def _compute_chunk(in_v, w_vec, out_v):
    iota = lax.iota(jnp.int32, _LANES)
    for g in range(_CH // _LANES):
        rows = iota + (g * _LANES)
        band = rows // 8
        sub = rows % 8
        cols = [jnp.full((_LANES,), j, jnp.int32) for j in range(_L)]
        vs = [plsc.load_gather(in_v, [band, sub, cols[j]]) for j in range(_L)]
        sgn = [(-1.0 if j % 2 == 0 else 1.0) for j in range(_L)]
        cm = [vs[j] * sgn[j] for j in range(_L)]
        m = cm[0]
        for j in range(1, _L):
            m = jnp.maximum(m, cm[j])
        es = [jnp.exp(cm[j] - m) for j in range(_L)]
        s = es[0]
        for j in range(1, _L):
            s = s + es[j]
        scale = w_vec / s
        for j in range(_L):
            plsc.store_scatter(out_v, [rows, cols[j]], es[j] * (scale * sgn[j]))


@functools.partial(
    pl.kernel,
    mesh=_mesh,
    compiler_params=pltpu.CompilerParams(needs_layout_passes=False),
    out_type=jax.ShapeDtypeStruct((_B, _P), jnp.float32),
    scratch_types=[
        pltpu.VMEM((_NB, 8, 128), jnp.float32),
        pltpu.VMEM((_NB, 8, 128), jnp.float32),
        pltpu.VMEM((_LANES,), jnp.float32),
        pltpu.VMEM((_CH, _P), jnp.float32),
        pltpu.VMEM((_CH, _P), jnp.float32),
        pltpu.SemaphoreType.DMA,
        pltpu.SemaphoreType.DMA,
        pltpu.SemaphoreType.DMA,
        pltpu.SemaphoreType.DMA,
    ],
)
def _sc_kernel(in3_hbm, w_hbm, out_hbm,
               in_v0, in_v1, w_v, ov0, ov1, osem0, osem1, isem0, isem1):
    wid = lax.axis_index("s") * _NC + lax.axis_index("c")
    row0 = wid * _ROWS_PER_W

    in_bufs = (in_v0, in_v1)
    out_bufs = (ov0, ov1)
    osems = (osem0, osem1)
    isems = (isem0, isem1)

    def _fetch(chunk, b):
        band0 = (row0 + chunk * _CH) // 8
        pltpu.async_copy(
            in3_hbm.at[pl.ds(band0, _NB), :, pl.ds(0, 128)],
            in_bufs[b], isems[b])

    def _fetch_wait(chunk, b):
        band0 = (row0 + chunk * _CH) // 8
        pltpu.make_async_copy(
            in3_hbm.at[pl.ds(band0, _NB), :, pl.ds(0, 128)],
            in_bufs[b], isems[b]).wait()

    _fetch(0, 0)
    _fetch(1, 1)
    w_copy = pltpu.make_async_copy(w_hbm, w_v, osem0)
    w_copy.start()

    zero = jnp.zeros((_LANES,), jnp.float32)

    def _zero_row(r, _):
        for c in range(_P // _LANES):
            ov0[r, pl.ds(c * _LANES, _LANES)] = zero
            ov1[r, pl.ds(c * _LANES, _LANES)] = zero
        return _

    lax.fori_loop(0, _CH, _zero_row, None)
    w_copy.wait()
    w_vec = w_v[...]

    def _pair(i, _):
        for b in range(2):
            chunk = i * 2 + b
            base = row0 + chunk * _CH

            _fetch_wait(chunk, b)

            @pl.when(i > 0)
            def _():
                pltpu.make_async_copy(
                    out_bufs[b], out_hbm.at[pl.ds(0, _CH)], osems[b]).wait()

            _compute_chunk(in_bufs[b], w_vec, out_bufs[b])
            pltpu.async_copy(
                out_bufs[b], out_hbm.at[pl.ds(base, _CH)], osems[b])

            @pl.when(chunk + 2 < _NCHUNK)
            def _():
                _fetch(chunk + 2, b)
        return _

    lax.fori_loop(0, _NCHUNK // 2, _pair, None)
    for b in range(2):
        pltpu.make_async_copy(
            out_bufs[b], out_hbm.at[pl.ds(0, _CH)], osems[b]).wait()


@jax.jit
def kernel(inputs, clause_weight):
    in3 = inputs.reshape(_B // 8, 8, _P)
    w16 = jnp.broadcast_to(clause_weight.reshape(()), (_LANES,))
    return _sc_kernel(in3, w16)


# speedup vs baseline: 1.7879x; 1.0034x over previous
import functools

import jax
import jax.numpy as jnp
from jax import lax
from jax.experimental import pallas as pl
from jax.experimental.pallas import tpu as pltpu
from jax.experimental.pallas import tpu_sc as plsc

_B, _P = 131072, 256
_L = 8
_NC, _NS, _LANES = 2, 16, 16
_NW = _NC * _NS
_ROWS_PER_W = _B // _NW
_CH = 128
_NB = _CH // 8
_NCHUNK = _ROWS_PER_W // _CH

_mesh = plsc.VectorSubcoreMesh(core_axis_name="c", subcore_axis_name="s")


def _compute_chunk(in_v, w_vec, out_v):
    iota = lax.iota(jnp.int32, _LANES)
    cols = [jnp.full((_LANES,), j, jnp.int32) for j in range(_L)]
    sgn = [(-1.0 if j % 2 == 0 else 1.0) for j in range(_L)]
    for g in range(_CH // _LANES):
        rows = iota + (g * _LANES)
        band = rows // 8
        sub = rows % 8
        vs = [plsc.load_gather(in_v, [band, sub, cols[j]]) for j in range(_L)]
        cm = [vs[j] * sgn[j] for j in range(_L)]
        m = cm[0]
        for j in range(1, _L):
            m = jnp.maximum(m, cm[j])
        es = [jnp.exp(cm[j] - m) for j in range(_L)]
        s = es[0]
        for j in range(1, _L):
            s = s + es[j]
        scale = w_vec / s
        nscale = -scale
        for j in range(_L):
            plsc.store_scatter(
                out_v, [rows, cols[j]],
                es[j] * (scale if sgn[j] > 0 else nscale))


@functools.partial(
    pl.kernel,
    mesh=_mesh,
    compiler_params=pltpu.CompilerParams(needs_layout_passes=False),
    out_type=jax.ShapeDtypeStruct((_B, _P), jnp.float32),
    scratch_types=[
        pltpu.VMEM((_NB, 8, 128), jnp.float32),
        pltpu.VMEM((_NB, 8, 128), jnp.float32),
        pltpu.VMEM((_LANES,), jnp.float32),
        pltpu.VMEM((_CH, _P), jnp.float32),
        pltpu.VMEM((_CH, _P), jnp.float32),
        pltpu.SemaphoreType.DMA,
        pltpu.SemaphoreType.DMA,
        pltpu.SemaphoreType.DMA,
        pltpu.SemaphoreType.DMA,
    ],
)
def _sc_kernel(in3_hbm, w_hbm, out_hbm,
               in_v0, in_v1, w_v, ov0, ov1, osem0, osem1, isem0, isem1):
    wid = lax.axis_index("s") * _NC + lax.axis_index("c")
    row0 = wid * _ROWS_PER_W

    in_bufs = (in_v0, in_v1)
    out_bufs = (ov0, ov1)
    osems = (osem0, osem1)
    isems = (isem0, isem1)

    def _fetch(chunk, b):
        band0 = (row0 + chunk * _CH) // 8
        pltpu.async_copy(
            in3_hbm.at[pl.ds(band0, _NB), :, pl.ds(0, 128)],
            in_bufs[b], isems[b])

    def _fetch_wait(chunk, b):
        band0 = (row0 + chunk * _CH) // 8
        pltpu.make_async_copy(
            in3_hbm.at[pl.ds(band0, _NB), :, pl.ds(0, 128)],
            in_bufs[b], isems[b]).wait()

    _fetch(0, 0)
    _fetch(1, 1)
    w_copy = pltpu.make_async_copy(w_hbm, w_v, osem0)
    w_copy.start()

    zero = jnp.zeros((_LANES,), jnp.float32)

    def _zero_row(r, _):
        for c in range(_P // _LANES):
            ov0[r, pl.ds(c * _LANES, _LANES)] = zero
            ov1[r, pl.ds(c * _LANES, _LANES)] = zero
        return _

    lax.fori_loop(0, _CH, _zero_row, None)
    w_copy.wait()
    w_vec = w_v[...]

    def _pair(i, _):
        for b in range(2):
            chunk = i * 2 + b
            base = row0 + chunk * _CH

            _fetch_wait(chunk, b)

            @pl.when(i > 0)
            def _():
                pltpu.make_async_copy(
                    out_bufs[b], out_hbm.at[pl.ds(0, _CH)], osems[b]).wait()

            _compute_chunk(in_bufs[b], w_vec, out_bufs[b])
            pltpu.async_copy(
                out_bufs[b], out_hbm.at[pl.ds(base, _CH)], osems[b])

            @pl.when(chunk + 2 < _NCHUNK)
            def _():
                _fetch(chunk + 2, b)
        return _

    lax.fori_loop(0, _NCHUNK // 2, _pair, None)
    for b in range(2):
        pltpu.make_async_copy(
            out_bufs[b], out_hbm.at[pl.ds(0, _CH)], osems[b]).wait()


@jax.jit
def kernel(inputs, clause_weight):
    in3 = inputs.reshape(_B // 8, 8, _P)
    w16 = jnp.broadcast_to(clause_weight.reshape(()), (_LANES,))
    return _sc_kernel(in3, w16)
